# baseline (speedup 1.0000x reference)
import functools

import jax
import jax.numpy as jnp
from jax import lax
from jax.experimental import pallas as pl
from jax.experimental.pallas import tpu as pltpu
from jax.experimental.pallas import tpu_sc as plsc

_B = 16384
_NC = 2
_NS = 16
_NW = _NC * _NS
_BPW = _B // _NW
_L = 16
_CHUNK = 128
_NCHUNK = _BPW // _CHUNK
_GPC = _CHUNK // _L

_DW = 64

_SFEATS = (
    (5, 0, 25),
    (3, 120, 30),
    (2, 156, 33),
    (1, 170, 35),
)
_STWORDS = 172
_STPAD = 192


def _sc_gather(emap_i32, tsml_i32, i_map, i_hour, i_month, i_dow, i_year):
    mesh = plsc.VectorSubcoreMesh(core_axis_name="c", subcore_axis_name="s")

    @functools.partial(
        pl.kernel,
        mesh=mesh,
        compiler_params=pltpu.CompilerParams(
            needs_layout_passes=False, use_tc_tiling_on_sc=False),
        out_type=jax.ShapeDtypeStruct((_NW, _BPW, _DW), jnp.int32),
        scratch_types=[
            pltpu.VMEM((_BPW,), jnp.int32),
            pltpu.VMEM((_BPW, _DW), jnp.int32),
            pltpu.VMEM((_STPAD,), jnp.int32),
            pltpu.VMEM((_BPW,), jnp.int32),
            pltpu.VMEM((_BPW,), jnp.int32),
            pltpu.VMEM((_BPW,), jnp.int32),
            pltpu.VMEM((_BPW,), jnp.int32),
            pltpu.SemaphoreType.DMA,
            pltpu.SemaphoreType.DMA,
            pltpu.SemaphoreType.DMA,
        ],
    )
    def k(emap_hbm, tsml_hbm, im_hbm, i1_hbm, i2_hbm, i3_hbm, i4_hbm,
          out_hbm, im_v, rows_v, tsml_v, i1_v, i2_v, i3_v, i4_v,
          sem, sem2, sem3):
        wid = lax.axis_index("s") * _NC + lax.axis_index("c")
        base = wid * _BPW
        pltpu.sync_copy(im_hbm.at[pl.ds(base, _BPW)], im_v)

        gathers = [
            pltpu.async_copy(
                emap_hbm.at[im_v.at[pl.ds(c * _CHUNK, _CHUNK)]],
                rows_v.at[pl.ds(c * _CHUNK, _CHUNK)],
                sem,
            )
            for c in range(_NCHUNK)
        ]

        small_copies = [
            pltpu.async_copy(tsml_hbm, tsml_v, sem2),
            pltpu.async_copy(i1_hbm.at[pl.ds(base, _BPW)], i1_v, sem2),
            pltpu.async_copy(i2_hbm.at[pl.ds(base, _BPW)], i2_v, sem2),
            pltpu.async_copy(i3_hbm.at[pl.ds(base, _BPW)], i3_v, sem2),
            pltpu.async_copy(i4_hbm.at[pl.ds(base, _BPW)], i4_v, sem2),
        ]
        for cp in small_copies:
            cp.wait()

        idx_refs = (i1_v, i2_v, i3_v, i4_v)

        def body(g, carry):
            b = g * _L
            rows16 = b + lax.iota(jnp.int32, _L)
            for (dim, toff, coff), iref in zip(_SFEATS, idx_refs):
                rows = iref[pl.ds(b, _L)]
                addr = rows * dim + toff if dim > 1 else rows + toff
                for j in range(dim):
                    v = plsc.load_gather(tsml_v, [addr + j if j else addr])
                    plsc.store_scatter(
                        rows_v,
                        [rows16, jnp.full((_L,), coff + j, jnp.int32)], v)
            return carry

        writes = []
        for c in range(_NCHUNK):
            gathers[c].wait()
            lax.fori_loop(c * _GPC, (c + 1) * _GPC, body, 0)
            writes.append(pltpu.async_copy(
                rows_v.at[pl.ds(c * _CHUNK, _CHUNK)],
                out_hbm.at[wid, pl.ds(c * _CHUNK, _CHUNK)],
                sem3,
            ))
        for wr in writes:
            wr.wait()

    return k(emap_i32, tsml_i32, i_map, i_hour, i_month, i_dow, i_year)


def _tc_mlp(ei, w1lo, w1hi, b1, w2, b2, w3, b3):
    bm = 2048
    nrows = ei.shape[0]

    def body(e_ref, w1lo_ref, w1hi_ref, b1_ref, w2_ref, b2_ref,
             w3_ref, b3_ref, o_ref):
        x = e_ref[...]
        w1lo_v = w1lo_ref[...]
        w1hi_v = w1hi_ref[...]
        a1s = []
        for h in (x[:, :_DW], x[:, _DW:]):
            lo = lax.bitcast_convert_type(
                h << 16, jnp.float32).astype(jnp.bfloat16)
            hi = lax.bitcast_convert_type(
                h & jnp.int32(-65536), jnp.float32).astype(jnp.bfloat16)
            a1s.append(
                jnp.dot(lo, w1lo_v, preferred_element_type=jnp.float32)
                + jnp.dot(hi, w1hi_v, preferred_element_type=jnp.float32))
        bm = x.shape[0]
        a1 = jnp.maximum(jnp.concatenate(a1s, axis=0) + b1_ref[...], 0.0)
        a2 = jnp.maximum(
            jnp.dot(a1, w2_ref[...], preferred_element_type=jnp.float32)
            + b2_ref[...], 0.0)
        z = jnp.dot(a2, w3_ref[...], preferred_element_type=jnp.float32)
        o = jax.nn.sigmoid(z + b3_ref[...])
        o_ref[...] = jnp.concatenate([o[:bm], o[bm:]], axis=1)

    return pl.pallas_call(
        body,
        grid=(nrows // bm,),
        in_specs=[
            pl.BlockSpec((bm, 2 * _DW), lambda i: (i, 0)),
            pl.BlockSpec((_DW, 100), lambda i: (0, 0)),
            pl.BlockSpec((_DW, 100), lambda i: (0, 0)),
            pl.BlockSpec((1, 100), lambda i: (0, 0)),
            pl.BlockSpec((100, 50), lambda i: (0, 0)),
            pl.BlockSpec((1, 50), lambda i: (0, 0)),
            pl.BlockSpec((50, 1), lambda i: (0, 0)),
            pl.BlockSpec((1, 1), lambda i: (0, 0)),
        ],
        out_specs=pl.BlockSpec((bm, 2), lambda i: (i, 0)),
        out_shape=jax.ShapeDtypeStruct((nrows, 2), jnp.float32),
    )(ei, w1lo, w1hi, b1, w2, b2, w3, b3)


def _pack_pairs(t_bf16):
    r, c = t_bf16.shape
    return lax.bitcast_convert_type(
        t_bf16.reshape(r, c // 2, 2), jnp.int32)


def kernel(mapidx, year, month, dow, hour, E_map, E_year, E_month, E_dow,
           E_hour, W1, b1, W2, b2, W3, b3):
    im = mapidx.reshape(-1).astype(jnp.int32)
    i1 = hour.reshape(-1).astype(jnp.int32)
    i2 = month.reshape(-1).astype(jnp.int32)
    i3 = dow.reshape(-1).astype(jnp.int32)
    i4 = year.reshape(-1).astype(jnp.int32)

    emap_i32 = _pack_pairs(
        jnp.pad(E_map.astype(jnp.bfloat16), ((0, 0), (0, 2 * _DW - 50))))
    hour_p = _pack_pairs(E_hour.astype(jnp.bfloat16))
    month_p = _pack_pairs(E_month.astype(jnp.bfloat16))
    dow_p = _pack_pairs(
        jnp.pad(E_dow.astype(jnp.bfloat16), ((0, 0), (0, 1))))
    year_p = _pack_pairs(
        jnp.pad(E_year.astype(jnp.bfloat16), ((0, 0), (0, 1))))
    tsml = jnp.concatenate([
        hour_p.reshape(-1), month_p.reshape(-1), dow_p.reshape(-1),
        year_p.reshape(-1),
        jnp.zeros((_STPAD - _STWORDS,), jnp.int32),
    ])

    out3 = _sc_gather(emap_i32, tsml, im, i1, i2, i3, i4)
    ei = out3.reshape(_B // 2, 2 * _DW)

    w1p = jnp.zeros((2 * _DW, 100), jnp.float32)
    w1p = w1p.at[0:50].set(W1[0:50])
    w1p = w1p.at[50:60].set(W1[60:70])
    w1p = w1p.at[60:66].set(W1[51:57])
    w1p = w1p.at[66:69].set(W1[57:60])
    w1p = w1p.at[70].set(W1[50])
    w1lo = w1p[0::2].astype(jnp.bfloat16)
    w1hi = w1p[1::2].astype(jnp.bfloat16)

    o2 = _tc_mlp(ei, w1lo, w1hi, b1.reshape(1, 100), W2,
                 b2.reshape(1, 50), W3, b3.reshape(1, 1))
    return o2.reshape(_B, 1)


# speedup vs baseline: 1.0049x; 1.0049x over previous
import functools

import jax
import jax.numpy as jnp
from jax import lax
from jax.experimental import pallas as pl
from jax.experimental.pallas import tpu as pltpu
from jax.experimental.pallas import tpu_sc as plsc

_B = 16384
_NC = 2
_NS = 16
_NW = _NC * _NS
_BPW = _B // _NW
_L = 16
_CHUNK = 128
_NCHUNK = _BPW // _CHUNK
_GPC = _CHUNK // _L

_DW = 64

_SFEATS = (
    (5, 0, 25),
    (3, 120, 30),
    (2, 156, 33),
    (1, 170, 35),
)
_STWORDS = 172
_STPAD = 192


def _sc_gather(emap_i32, tsml_i32, i_map, i_hour, i_month, i_dow, i_year):
    mesh = plsc.VectorSubcoreMesh(core_axis_name="c", subcore_axis_name="s")

    @functools.partial(
        pl.kernel,
        mesh=mesh,
        compiler_params=pltpu.CompilerParams(
            needs_layout_passes=False, use_tc_tiling_on_sc=False),
        out_type=jax.ShapeDtypeStruct((_NW, _BPW, _DW), jnp.int32),
        scratch_types=[
            pltpu.VMEM((_BPW,), jnp.int32),
            pltpu.VMEM((_BPW, _DW), jnp.int32),
            pltpu.VMEM((_STPAD,), jnp.int32),
            pltpu.VMEM((_BPW,), jnp.int32),
            pltpu.VMEM((_BPW,), jnp.int32),
            pltpu.VMEM((_BPW,), jnp.int32),
            pltpu.VMEM((_BPW,), jnp.int32),
            pltpu.SemaphoreType.DMA,
            pltpu.SemaphoreType.DMA,
            pltpu.SemaphoreType.DMA,
        ],
    )
    def k(emap_hbm, tsml_hbm, im_hbm, i1_hbm, i2_hbm, i3_hbm, i4_hbm,
          out_hbm, im_v, rows_v, tsml_v, i1_v, i2_v, i3_v, i4_v,
          sem, sem2, sem3):
        wid = lax.axis_index("s") * _NC + lax.axis_index("c")
        base = wid * _BPW
        pltpu.sync_copy(im_hbm.at[pl.ds(base, _BPW)], im_v)

        gathers = [
            pltpu.async_copy(
                emap_hbm.at[im_v.at[pl.ds(c * _CHUNK, _CHUNK)]],
                rows_v.at[pl.ds(c * _CHUNK, _CHUNK)],
                sem,
            )
            for c in range(_NCHUNK)
        ]

        small_copies = [
            pltpu.async_copy(tsml_hbm, tsml_v, sem2),
            pltpu.async_copy(i1_hbm.at[pl.ds(base, _BPW)], i1_v, sem2),
            pltpu.async_copy(i2_hbm.at[pl.ds(base, _BPW)], i2_v, sem2),
            pltpu.async_copy(i3_hbm.at[pl.ds(base, _BPW)], i3_v, sem2),
            pltpu.async_copy(i4_hbm.at[pl.ds(base, _BPW)], i4_v, sem2),
        ]
        for cp in small_copies:
            cp.wait()

        idx_refs = (i1_v, i2_v, i3_v, i4_v)

        def body(g, carry):
            b = g * _L
            rows16 = b + lax.iota(jnp.int32, _L)
            for (dim, toff, coff), iref in zip(_SFEATS, idx_refs):
                rows = iref[pl.ds(b, _L)]
                addr = rows * dim + toff if dim > 1 else rows + toff
                for j in range(dim):
                    v = plsc.load_gather(tsml_v, [addr + j if j else addr])
                    plsc.store_scatter(
                        rows_v,
                        [rows16, jnp.full((_L,), coff + j, jnp.int32)], v)
            return carry

        writes = []
        for c in range(_NCHUNK):
            gathers[c].wait()
            lax.fori_loop(c * _GPC, (c + 1) * _GPC, body, 0)
            writes.append(pltpu.async_copy(
                rows_v.at[pl.ds(c * _CHUNK, _CHUNK)],
                out_hbm.at[wid, pl.ds(c * _CHUNK, _CHUNK)],
                sem3,
            ))
        for wr in writes:
            wr.wait()

    return k(emap_i32, tsml_i32, i_map, i_hour, i_month, i_dow, i_year)


def _tc_mlp(ei, w1lo, w1hi, b1, w2, b2, w3, b3):
    bm = 4096
    nrows = ei.shape[0]

    def body(e_ref, w1lo_ref, w1hi_ref, b1_ref, w2_ref, b2_ref,
             w3_ref, b3_ref, o_ref):
        x = e_ref[...]
        w1lo_v = w1lo_ref[...]
        w1hi_v = w1hi_ref[...]
        a1s = []
        for h in (x[:, :_DW], x[:, _DW:]):
            lo = lax.bitcast_convert_type(
                h << 16, jnp.float32).astype(jnp.bfloat16)
            hi = lax.bitcast_convert_type(
                h & jnp.int32(-65536), jnp.float32).astype(jnp.bfloat16)
            a1s.append(
                jnp.dot(lo, w1lo_v, preferred_element_type=jnp.float32)
                + jnp.dot(hi, w1hi_v, preferred_element_type=jnp.float32))
        bm = x.shape[0]
        a1 = jnp.maximum(jnp.concatenate(a1s, axis=0) + b1_ref[...], 0.0)
        a2 = jnp.maximum(
            jnp.dot(a1, w2_ref[...], preferred_element_type=jnp.float32)
            + b2_ref[...], 0.0)
        z = jnp.dot(a2, w3_ref[...], preferred_element_type=jnp.float32)
        o = jax.nn.sigmoid(z + b3_ref[...])
        o_ref[...] = jnp.concatenate([o[:bm], o[bm:]], axis=1)

    return pl.pallas_call(
        body,
        grid=(nrows // bm,),
        in_specs=[
            pl.BlockSpec((bm, 2 * _DW), lambda i: (i, 0)),
            pl.BlockSpec((_DW, 100), lambda i: (0, 0)),
            pl.BlockSpec((_DW, 100), lambda i: (0, 0)),
            pl.BlockSpec((1, 100), lambda i: (0, 0)),
            pl.BlockSpec((100, 50), lambda i: (0, 0)),
            pl.BlockSpec((1, 50), lambda i: (0, 0)),
            pl.BlockSpec((50, 1), lambda i: (0, 0)),
            pl.BlockSpec((1, 1), lambda i: (0, 0)),
        ],
        out_specs=pl.BlockSpec((bm, 2), lambda i: (i, 0)),
        out_shape=jax.ShapeDtypeStruct((nrows, 2), jnp.float32),
    )(ei, w1lo, w1hi, b1, w2, b2, w3, b3)


def _pack_pairs(t_bf16):
    r, c = t_bf16.shape
    return lax.bitcast_convert_type(
        t_bf16.reshape(r, c // 2, 2), jnp.int32)


def kernel(mapidx, year, month, dow, hour, E_map, E_year, E_month, E_dow,
           E_hour, W1, b1, W2, b2, W3, b3):
    im = mapidx.reshape(-1).astype(jnp.int32)
    i1 = hour.reshape(-1).astype(jnp.int32)
    i2 = month.reshape(-1).astype(jnp.int32)
    i3 = dow.reshape(-1).astype(jnp.int32)
    i4 = year.reshape(-1).astype(jnp.int32)

    emap_i32 = _pack_pairs(
        jnp.pad(E_map.astype(jnp.bfloat16), ((0, 0), (0, 2 * _DW - 50))))
    hour_p = _pack_pairs(E_hour.astype(jnp.bfloat16))
    month_p = _pack_pairs(E_month.astype(jnp.bfloat16))
    dow_p = _pack_pairs(
        jnp.pad(E_dow.astype(jnp.bfloat16), ((0, 0), (0, 1))))
    year_p = _pack_pairs(
        jnp.pad(E_year.astype(jnp.bfloat16), ((0, 0), (0, 1))))
    tsml = jnp.concatenate([
        hour_p.reshape(-1), month_p.reshape(-1), dow_p.reshape(-1),
        year_p.reshape(-1),
        jnp.zeros((_STPAD - _STWORDS,), jnp.int32),
    ])

    out3 = _sc_gather(emap_i32, tsml, im, i1, i2, i3, i4)
    ei = out3.reshape(_B // 2, 2 * _DW)

    w1p = jnp.zeros((2 * _DW, 100), jnp.float32)
    w1p = w1p.at[0:50].set(W1[0:50])
    w1p = w1p.at[50:60].set(W1[60:70])
    w1p = w1p.at[60:66].set(W1[51:57])
    w1p = w1p.at[66:69].set(W1[57:60])
    w1p = w1p.at[70].set(W1[50])
    w1lo = w1p[0::2].astype(jnp.bfloat16)
    w1hi = w1p[1::2].astype(jnp.bfloat16)

    o2 = _tc_mlp(ei, w1lo, w1hi, b1.reshape(1, 100), W2,
                 b2.reshape(1, 50), W3, b3.reshape(1, 1))
    return o2.reshape(_B, 1)
